# baseline (speedup 1.0000x reference)
import functools

import jax
import jax.numpy as jnp
from jax import lax
from jax.experimental import pallas as pl
from jax.experimental.pallas import tpu as pltpu

K = 1024
D = 64
JB = 8192


def _vq_block(x_ref, w_ref, q_ref, loss_ref, cnt_ref):
    xb = x_ref[...]
    w = w_ref[...]
    w2 = w * (-2.0)
    mm2 = lax.dot_general(xb, w2, (((1,), (1,)), ((), ())),
                          preferred_element_type=jnp.float32)
    xsq = jnp.sum(xb * xb, axis=1, keepdims=True)
    wsq = jnp.sum(w * w, axis=1)
    d = (xsq + wsq[None, :]) + mm2
    m = jnp.min(d, axis=1, keepdims=True)
    ks = lax.broadcasted_iota(jnp.int32, d.shape, 1)
    nearest = jnp.min(jnp.where(d == m, ks, K), axis=1)
    oh = (ks == nearest[:, None]).astype(jnp.float32)
    qc = lax.dot_general(w, oh, (((0,), (1,)), ((), ())),
                         preferred_element_type=jnp.float32)
    for i in range(JB // 1024):
        q_ref[i] = qc[:, i * 1024:(i + 1) * 1024]
    loss_ref[...] = jnp.broadcast_to(jnp.sum(m), (1, 1, 128))
    cnt_ref[...] = jnp.sum(oh, axis=0)[None, None, :]


def kernel(inputs, W, beta):
    B, C, H, Wd = inputs.shape
    N = B * H * Wd
    nb = N // JB
    x = jnp.transpose(inputs, (0, 2, 3, 1)).reshape(N, D)
    q, lsum, cnt = pl.pallas_call(
        _vq_block,
        grid=(nb,),
        in_specs=[
            pl.BlockSpec((JB, D), lambda j: (j, 0)),
            pl.BlockSpec((K, D), lambda j: (0, 0)),
        ],
        out_specs=[
            pl.BlockSpec((JB // 1024, D, H * Wd), lambda j: (j, 0, 0)),
            pl.BlockSpec((1, 1, 128), lambda j: (j, 0, 0)),
            pl.BlockSpec((1, 1, K), lambda j: (j, 0, 0)),
        ],
        out_shape=[
            jax.ShapeDtypeStruct((B, D, H * Wd), jnp.float32),
            jax.ShapeDtypeStruct((nb, 1, 128), jnp.float32),
            jax.ShapeDtypeStruct((nb, 1, K), jnp.float32),
        ],
    )(x, W)
    loss_mean = jnp.sum(lsum[:, 0, 0]) / (N * D)
    loss = loss_mean + beta * loss_mean
    e_mean = jnp.sum(cnt[:, 0, :], axis=0) / N
    perplexity = jnp.exp(-jnp.sum(e_mean * jnp.log(e_mean + 1e-10)))
    quantized_out = q.reshape(B, C, H, Wd)
    return (loss, quantized_out, perplexity)
